# baseline (speedup 1.0000x reference)
import jax
import jax.numpy as jnp
from jax import lax
from jax.experimental import pallas as pl
from jax.experimental.pallas import tpu as pltpu
from jax.experimental.pallas import tpu_sc as plsc

_B = 4096
_L = 200
_EMB = 64
_PAD = 128
_N = _B * _L
_NW = 32
_R = _N // _NW
_C = _L
_NCH = _R // _C
_NBUF = 4
_NPB = 2
_S = 5
_CS = _C // _S


def _emb_kernel(wt, pt, ids, pids, out, ptab,
                w0, w1, w2, w3, p0, p1,
                i0, i1, i2, i3, q0, q1, q2, q3,
                gw0, gw1, gw2, gw3, gp0, gp1,
                os0, os1, os2, os3, is0, is1, is2, is3):
    wbufs = [w0, w1, w2, w3]
    pbufs = [p0, p1]
    islt = [i0, i1, i2, i3]
    qslt = [q0, q1, q2, q3]
    gw = [gw0, gw1, gw2, gw3]
    gp = [gp0, gp1]
    osm = [os0, os1, os2, os3]
    ism = [is0, is1, is2, is3]

    wid = lax.axis_index("s") * 2 + lax.axis_index("c")
    base = wid * _R
    base_b = wid * _NCH

    @pl.when(lax.axis_index("s") == 0)
    def _():
        pltpu.sync_copy(pt, ptab)
    plsc.subcore_barrier()

    def idx_descs(k, j):
        sl = pl.ds(base + k * _C, _C)
        return (pltpu.make_async_copy(ids.at[sl], islt[j], ism[j]),
                pltpu.make_async_copy(pids.at[sl], qslt[j], ism[j]))

    def word_descs(b, j):
        return [pltpu.make_async_copy(
                    wt.at[islt[j].at[pl.ds(s * _CS, _CS)]],
                    wbufs[b].at[pl.ds(s * _CS, _CS)], gw[b])
                for s in range(_S)]

    def pos_desc(pb, j):
        return pltpu.make_async_copy(ptab.at[qslt[j]], pbufs[pb], gp[pb])

    def out_desc(k, b):
        return pltpu.make_async_copy(wbufs[b], out.at[base_b + k], osm[b])

    for k in range(_NBUF):
        da, db = idx_descs(k, k)
        da.start()
        db.start()
    for b in range(2):
        da, db = idx_descs(b, b)
        da.wait()
        db.wait()
        for d in word_descs(b, b):
            d.start()
    pos_desc(0, 0).start()

    @pl.loop(0, _NCH // _NBUF)
    def _(ko):
        for b in range(_NBUF):
            k = ko * _NBUF + b
            bn = (b + 2) % _NBUF
            pb = b % _NPB

            @pl.when(k + 2 < _NCH)
            def _():
                da, db = idx_descs(k + 2, bn)
                da.wait()
                db.wait()

                @pl.when(k >= 2)
                def _():
                    out_desc(k - 2, bn).wait()
                for d in word_descs(bn, bn):
                    d.start()

            for d in word_descs(b, b):
                d.wait()
            pos_desc(pb, b).wait()

            @pl.when(k + 4 < _NCH)
            def _():
                da, db = idx_descs(k + 4, b)
                da.start()
                db.start()

            @pl.loop(0, _C)
            def _(r):
                for c in range(_EMB // 16):
                    sl = pl.ds(c * 16, 16)
                    wbufs[b][r, sl] += pbufs[pb][r, sl]

            out_desc(k, b).start()

            @pl.when(k + 1 < _NCH)
            def _():
                pos_desc((pb + 1) % _NPB, (b + 1) % _NBUF).start()

    for b in range(_NBUF):
        out_desc(_NCH - _NBUF + b, b).wait()


def kernel(input_ids, position_ids, word_table, pos_table):
    ids = input_ids.astype(jnp.int32).reshape(_N)
    pids = position_ids.astype(jnp.int32).reshape(_N)
    wt_pad = jnp.pad(word_table, ((0, 0), (0, _PAD - _EMB)))
    mesh = plsc.VectorSubcoreMesh(core_axis_name="c", subcore_axis_name="s")
    run = pl.kernel(
        _emb_kernel,
        out_type=jax.ShapeDtypeStruct((_B, _L, _PAD), jnp.float32),
        mesh=mesh,
        scratch_types=(
            [pltpu.VMEM_SHARED((_L, _EMB), jnp.float32)]
            + [pltpu.VMEM((_C, _PAD), jnp.float32)] * _NBUF
            + [pltpu.VMEM((_C, _EMB), jnp.float32)] * _NPB
            + [pltpu.VMEM((_C,), jnp.int32)] * (2 * _NBUF)
            + [pltpu.SemaphoreType.DMA] * (2 * _NBUF + _NPB + _NBUF)
        ),
        compiler_params=pltpu.CompilerParams(use_tc_tiling_on_sc=False),
    )
    return run(wt_pad, pos_table, ids, pids)[:, :, :_EMB]


# speedup vs baseline: 1.3547x; 1.3547x over previous
import jax
import jax.numpy as jnp
from jax import lax
from jax.experimental import pallas as pl
from jax.experimental.pallas import tpu as pltpu
from jax.experimental.pallas import tpu_sc as plsc

_B = 4096
_L = 200
_EMB = 64
_N = _B * _L
_NW = 32
_R = _N // _NW
_C = 200
_NCH = _R // _C
_NBUF = 4


def _emb_kernel(wt, pt, ids, pids, out, ptab,
                w0, w1, w2, w3, p0, p1, p2, p3,
                i0, i1, i2, i3, q0, q1, q2, q3,
                gw0, gw1, gw2, gw3, gp0, gp1, gp2, gp3,
                os0, os1, os2, os3, is0, is1, is2, is3):
    wbufs = [w0, w1, w2, w3]
    pbufs = [p0, p1, p2, p3]
    islt = [i0, i1, i2, i3]
    qslt = [q0, q1, q2, q3]
    gw = [gw0, gw1, gw2, gw3]
    gp = [gp0, gp1, gp2, gp3]
    osm = [os0, os1, os2, os3]
    ism = [is0, is1, is2, is3]

    wid = lax.axis_index("s") * 2 + lax.axis_index("c")
    base = wid * _R

    @pl.when(lax.axis_index("s") == 0)
    def _():
        pltpu.sync_copy(pt, ptab)
    plsc.subcore_barrier()

    def idx_descs(k, j):
        sl = pl.ds(base + k * _C, _C)
        return (pltpu.make_async_copy(ids.at[sl], islt[j], ism[j]),
                pltpu.make_async_copy(pids.at[sl], qslt[j], ism[j]))

    _S = 5
    _CS = _C // _S

    def gather_descs(b):
        subs = [pltpu.make_async_copy(
                    wt.at[islt[b].at[pl.ds(s * _CS, _CS)]],
                    wbufs[b].at[pl.ds(s * _CS, _CS)], gw[b])
                for s in range(_S)]
        subs.append(pltpu.make_async_copy(ptab.at[qslt[b]], pbufs[b], gp[b]))
        return subs

    base_b = wid * (_R // _C)

    def out_desc(k, b):
        return pltpu.make_async_copy(
            wbufs[b], out.at[base_b + k, :, pl.ds(0, _EMB)], osm[b])

    for k in range(4):
        da, db = idx_descs(k, k)
        da.start()
        db.start()
    for b in range(2):
        da, db = idx_descs(b, b)
        da.wait()
        db.wait()
        for d in gather_descs(b):
            d.start()

    @pl.loop(0, _NCH // _NBUF)
    def _(ko):
        for b in range(_NBUF):
            k = ko * _NBUF + b
            bn = (b + 2) % _NBUF

            @pl.when(k + 2 < _NCH)
            def _():
                da, db = idx_descs(k + 2, bn)
                da.wait()
                db.wait()

                @pl.when(k >= 2)
                def _():
                    out_desc(k - 2, bn).wait()
                for d in gather_descs(bn):
                    d.start()

            for d in gather_descs(b):
                d.wait()

            @pl.when(k + 4 < _NCH)
            def _():
                da, db = idx_descs(k + 4, b)
                da.start()
                db.start()

            @pl.loop(0, _C)
            def _(r):
                for c in range(_EMB // 16):
                    sl = pl.ds(c * 16, 16)
                    wbufs[b][r, sl] += pbufs[b][r, sl]

            out_desc(k, b).start()

    for b in range(_NBUF):
        out_desc(_NCH - _NBUF + b, b).wait()


def kernel(input_ids, position_ids, word_table, pos_table):
    ids = input_ids.astype(jnp.int32).reshape(_N)
    pids = position_ids.astype(jnp.int32).reshape(_N)
    mesh = plsc.VectorSubcoreMesh(core_axis_name="c", subcore_axis_name="s")
    run = pl.kernel(
        _emb_kernel,
        out_type=jax.ShapeDtypeStruct((_B, _L, 128), jnp.float32),
        mesh=mesh,
        scratch_types=(
            [pltpu.VMEM_SHARED((_L, _EMB), jnp.float32)]
            + [pltpu.VMEM((_C, _EMB), jnp.float32)] * (2 * _NBUF)
            + [pltpu.VMEM((_C,), jnp.int32)] * (2 * _NBUF)
            + [pltpu.SemaphoreType.DMA] * (4 * _NBUF)
        ),
        compiler_params=pltpu.CompilerParams(use_tc_tiling_on_sc=False),
    )
    return run(word_table, pos_table, ids, pids)[:, :, :_EMB]
